# Initial kernel scaffold; baseline (speedup 1.0000x reference)
import jax
import jax.numpy as jnp
from jax.experimental import pallas as pl


def kernel(inputs, row_indices, col_indices):
    raise NotImplementedError("write your pallas kernel here")


# speedup vs baseline: 40.1926x; 40.1926x over previous
import jax
import jax.numpy as jnp
from jax import lax
from jax.experimental import pallas as pl
from jax.experimental.pallas import tpu as pltpu
from jax.experimental.pallas import tpu_sc as plsc

B = 32
N = 500000
GR = 1024
GC = 1024
NC = 2
NS = 16
NW = NC * NS
ROWS_PER_W = GR // NW
CELLS_PER_W = ROWS_PER_W * GC
CH = 4000
NCHUNK = N // CH
NVEC = CH // 16
ZPAD = 512
SCH = 2000
NSCH = N // SCH


def _body(in_hbm, row_hbm, col_hbm, out_hbm,
          rbuf, cbuf, owner, plane, zbuf, sbuf, spin, sem):
    c = lax.axis_index("c")
    s = lax.axis_index("s")
    wid = s * NC + c
    base = wid * CELLS_PER_W

    def init_row(j, _):
        owner[pl.ds(j * 16, 16)] = jnp.full((16,), -1, jnp.int32)
        return 0
    lax.fori_loop(0, CELLS_PER_W // 16, init_row, 0)

    def chunk_body(ii, _):
        i = NCHUNK - 1 - ii
        off = i * CH
        pltpu.sync_copy(row_hbm.at[pl.ds(off, CH)], rbuf)
        pltpu.sync_copy(col_hbm.at[pl.ds(off, CH)], cbuf)

        def vec_body(vv, _):
            v = NVEC - 1 - vv
            r = rbuf[pl.ds(v * 16, 16)]
            cc = cbuf[pl.ds(v * 16, 16)]
            local = r * GC + cc - base
            m = local.astype(jnp.uint32) < jnp.uint32(CELLS_PER_W)
            lsafe = jnp.bitwise_and(local, CELLS_PER_W - 1)
            n_vec = off + v * 16 + lax.iota(jnp.int32, 16)
            plsc.store_scatter(owner, [lsafe], n_vec, mask=m)
            return 0
        lax.fori_loop(0, NVEC, vec_body, 0)
        return 0
    lax.fori_loop(0, NCHUNK, chunk_body, 0)

    def fix_row(j, _):
        o = owner[pl.ds(j * 16, 16)]
        cell = j * 16 + lax.iota(jnp.int32, 16)
        idx = jnp.where(o < 0, N + jnp.bitwise_and(cell, ZPAD - 1), o)
        owner[pl.ds(j * 16, 16)] = idx
        return 0
    lax.fori_loop(0, CELLS_PER_W // 16, fix_row, 0)

    @pl.when(s == 0)
    def _():
        def zrow(k, _):
            zbuf[pl.ds(k * 16, 16)] = jnp.zeros((16,), jnp.float32)
            return 0
        lax.fori_loop(0, ZPAD // 16, zrow, 0)
        pltpu.sync_copy(zbuf, spin.at[pl.ds(N, ZPAD)])

    def batch_body(b, _):
        plsc.subcore_barrier()

        def stage_chunk(k, _):
            idx = s + 16 * k

            @pl.when(idx < NSCH)
            def _():
                off = idx * SCH
                pltpu.sync_copy(in_hbm.at[pl.ds(b * N + off, SCH)], sbuf)
                pltpu.sync_copy(sbuf, spin.at[pl.ds(off, SCH)])
            return 0
        lax.fori_loop(0, (NSCH + NS - 1) // NS, stage_chunk, 0)
        plsc.subcore_barrier()

        pltpu.async_copy(spin.at[owner], plane, sem).wait()
        pltpu.sync_copy(
            plane,
            out_hbm.at[pl.ds(b * GR * GC + wid * CELLS_PER_W, CELLS_PER_W)])
        return 0
    lax.fori_loop(0, B, batch_body, 0)


_regrid = pl.kernel(
    _body,
    out_type=jax.ShapeDtypeStruct((B * GR * GC,), jnp.float32),
    mesh=plsc.VectorSubcoreMesh(
        core_axis_name="c", subcore_axis_name="s",
        num_cores=NC, num_subcores=NS),
    compiler_params=pltpu.CompilerParams(needs_layout_passes=False),
    scratch_types=[
        pltpu.VMEM((CH,), jnp.int32),
        pltpu.VMEM((CH,), jnp.int32),
        pltpu.VMEM((CELLS_PER_W,), jnp.int32),
        pltpu.VMEM((CELLS_PER_W,), jnp.float32),
        pltpu.VMEM((ZPAD,), jnp.float32),
        pltpu.VMEM((SCH,), jnp.float32),
        pltpu.VMEM_SHARED((N + ZPAD,), jnp.float32),
        pltpu.SemaphoreType.DMA,
    ],
)


def kernel(inputs, row_indices, col_indices):
    out = _regrid(inputs.reshape(-1), row_indices, col_indices)
    return out.reshape(B, GR, GC, 1)
